# Initial kernel scaffold; baseline (speedup 1.0000x reference)
import jax
import jax.numpy as jnp
from jax.experimental import pallas as pl


def kernel(feat, edge_index0, edge_index1, num_recv_dst, num_send_dst, weight, bias):
    raise NotImplementedError("write your pallas kernel here")


# speedup vs baseline: 3.1720x; 3.1720x over previous
import functools

import jax
import jax.numpy as jnp
from jax import lax
from jax.experimental import pallas as pl
from jax.experimental.pallas import tpu as pltpu
from jax.experimental.pallas import tpu_sc as plsc

_NC = 2
_NS = 16
_LANES = 16
_CHUNK = 128

_F32 = jnp.float32
_I32 = jnp.int32


def _rsqrt16(v):
    x = jnp.maximum(v, 1.0)
    i = lax.bitcast_convert_type(x, _I32)
    i = jnp.int32(0x5F3759DF) - lax.shift_right_arithmetic(i, 1)
    y = lax.bitcast_convert_type(i, _F32)
    for _ in range(3):
        y = y * (1.5 - 0.5 * x * y * y)
    return y


def _row_partition(nv):
    rpt = (-(-nv // _NS) + _LANES - 1) // _LANES * _LANES
    rlast = nv - rpt * (_NS - 1)
    assert rlast > 0 and rlast % _LANES == 0 and rpt % _LANES == 0
    return rpt, rlast


def _deg_norms(src0, dst0, dst1, nv):
    e = src0.shape[0]
    assert e % _NS == 0
    ept = e // _NS
    nfull = ept // _CHUNK
    tail = ept - nfull * _CHUNK
    assert tail % 8 == 0
    rpt, rlast = _row_partition(nv)
    mesh = plsc.VectorSubcoreMesh(
        core_axis_name="c", subcore_axis_name="s",
        num_cores=_NC, num_subcores=_NS)

    @functools.partial(
        pl.kernel,
        out_type=[jax.ShapeDtypeStruct((nv, _LANES), _F32)] * 3,
        mesh=mesh,
        scratch_types=[
            pltpu.VMEM_SHARED((nv, _LANES), _F32),
            pltpu.VMEM_SHARED((nv, _LANES), _F32),
            pltpu.VMEM((rpt, _LANES), _F32),
            pltpu.VMEM((rpt, _LANES), _F32),
            pltpu.VMEM((_CHUNK,), _I32),
            pltpu.VMEM((max(tail, 8),), _I32),
            pltpu.VMEM((_CHUNK, _LANES), _F32),
        ],
        compiler_params=pltpu.CompilerParams(use_tc_tiling_on_sc=False),
    )
    def deg_k(src0_ref, dst0_ref, dst1_ref, nl_ref, nr0_ref, nr1_ref,
              acc_a, acc_b, ext, nbuf, idx, idxt, ones):
        c = lax.axis_index("c")
        s = lax.axis_index("s")

        def fill_ones(r, _):
            ones[r, :] = jnp.ones((_LANES,), _F32)
            return 0
        lax.fori_loop(0, _CHUNK, fill_ones, 0)

        def fill_zero(r, _):
            ext[r, :] = jnp.zeros((_LANES,), _F32)
            return 0
        lax.fori_loop(0, rpt, fill_zero, 0)

        def two_path(fn):
            @pl.when(s < _NS - 1)
            def _():
                fn(s * rpt, rpt)

            @pl.when(s == _NS - 1)
            def _():
                fn((_NS - 1) * rpt, rlast)

        def zero_acc(acc):
            def z(r0, nr):
                pltpu.sync_copy(ext.at[pl.ds(0, nr)], acc.at[pl.ds(r0, nr)])
            two_path(z)

        zero_acc(acc_a)
        zero_acc(acc_b)
        plsc.subcore_barrier()

        def scatter_ones(idx_ref, acc):
            base = s * ept

            def body(k, _):
                off = base + k * _CHUNK
                pltpu.sync_copy(idx_ref.at[pl.ds(off, _CHUNK)], idx)
                pltpu.sync_copy(ones, acc.at[idx], add=True)
                return 0
            lax.fori_loop(0, nfull, body, 0)
            if tail:
                off = base + nfull * _CHUNK
                pltpu.sync_copy(idx_ref.at[pl.ds(off, tail)],
                                idxt.at[pl.ds(0, tail)])
                pltpu.sync_copy(ones.at[pl.ds(0, tail)], acc.at[idxt],
                                add=True)

        @pl.when(c == 0)
        def _():
            scatter_ones(src0_ref, acc_a)
            scatter_ones(dst0_ref, acc_b)

        @pl.when(c == 1)
        def _():
            scatter_ones(dst1_ref, acc_a)

        plsc.subcore_barrier()

        def extract(acc, out_ref):
            def ext_rows(r0, nr):
                pltpu.sync_copy(acc.at[pl.ds(r0, nr)], ext.at[pl.ds(0, nr)])

                def row(r, _):
                    nbuf[r, :] = _rsqrt16(ext[r, :])
                    return 0
                lax.fori_loop(0, nr, row, 0)
                pltpu.sync_copy(nbuf.at[pl.ds(0, nr)],
                                out_ref.at[pl.ds(r0, nr)])
            two_path(ext_rows)

        @pl.when(c == 0)
        def _():
            extract(acc_a, nl_ref)
            extract(acc_b, nr0_ref)

        @pl.when(c == 1)
        def _():
            extract(acc_a, nr1_ref)

    return deg_k(src0, dst0, dst1)


def _project(x, w, nl):
    n, d_in = x.shape
    d_out = w.shape[1]
    half = d_out // 2
    bm = 400
    assert n % bm == 0

    def body(x_ref, w_ref, s_ref, y0_ref, y1_ref):
        y = jnp.dot(x_ref[...], w_ref[...],
                    preferred_element_type=jnp.float32)
        y = y * s_ref[...]
        y0_ref[...] = y[:, :half]
        y1_ref[...] = y[:, half:]

    return pl.pallas_call(
        body,
        grid=(n // bm,),
        in_specs=[
            pl.BlockSpec((bm, d_in), lambda i: (i, 0)),
            pl.BlockSpec((d_in, d_out), lambda i: (0, 0)),
            pl.BlockSpec((bm, 1), lambda i: (i, 0)),
        ],
        out_specs=[
            pl.BlockSpec((bm, half), lambda i: (i, 0)),
            pl.BlockSpec((bm, half), lambda i: (i, 0)),
        ],
        out_shape=[jax.ShapeDtypeStruct((n, half), _F32)] * 2,
    )(x, w, nl)


def _spconv(y0, y1, src0, dst0, src1, dst1, nr0, nr1, b0, b1):
    nv, half = y0.shape
    e0 = src0.shape[0]
    e1 = src1.shape[0]
    assert e0 % _NS == 0 and e1 % _NS == 0
    ept0 = e0 // _NS
    ept1 = e1 // _NS
    nfull0, tail0 = ept0 // _CHUNK, ept0 % _CHUNK
    nfull1, tail1 = ept1 // _CHUNK, ept1 % _CHUNK
    assert tail0 == tail1 and tail0 % 8 == 0
    tail = tail0
    rpt, rlast = _row_partition(nv)
    zrows = 80
    assert rpt % zrows == 0 and rlast % zrows == 0
    mesh = plsc.VectorSubcoreMesh(
        core_axis_name="c", subcore_axis_name="s",
        num_cores=_NC, num_subcores=_NS)

    @functools.partial(
        pl.kernel,
        out_type=[jax.ShapeDtypeStruct((nv, half), _F32)] * 4,
        mesh=mesh,
        scratch_types=[
            pltpu.VMEM_SHARED((nv, half), _F32),
            pltpu.VMEM((zrows, half), _F32),
            pltpu.VMEM((_CHUNK, half), _F32),
            pltpu.VMEM((max(tail, 8), half), _F32),
            pltpu.VMEM((zrows, half), _F32),
            pltpu.VMEM((_CHUNK,), _I32),
            pltpu.VMEM((_CHUNK,), _I32),
            pltpu.VMEM((max(tail, 8),), _I32),
            pltpu.VMEM((max(tail, 8),), _I32),
            pltpu.VMEM((zrows, _LANES), _F32),
            pltpu.VMEM((half,), _F32),
            pltpu.SemaphoreType.DMA,
        ],
        compiler_params=pltpu.CompilerParams(use_tc_tiling_on_sc=False),
    )
    def conv_k(y0_ref, y1_ref, s0_ref, d0_ref, s1_ref, d1_ref,
               nr0_ref, nr1_ref, b0_ref, b1_ref,
               hv0_ref, hv1_ref, rst0_ref, rst1_ref,
               acc, stage, gbuf, tgbuf, zbuf, sidx, didx, tsidx, tdidx,
               nbuf, bbuf, sem):
        c = lax.axis_index("c")
        s = lax.axis_index("s")
        ng = half // _LANES

        def fill_zero(r, _):
            def zc(g, _):
                zbuf[r, pl.ds(g * _LANES, _LANES)] = jnp.zeros((_LANES,),
                                                               _F32)
                return 0
            lax.fori_loop(0, ng, zc, 0)
            return 0
        lax.fori_loop(0, zrows, fill_zero, 0)

        def two_path(fn):
            @pl.when(s < _NS - 1)
            def _():
                fn(s * rpt, rpt)

            @pl.when(s == _NS - 1)
            def _():
                fn((_NS - 1) * rpt, rlast)

        def zero_acc(r0, nr):
            for j in range(nr // zrows):
                pltpu.sync_copy(zbuf, acc.at[pl.ds(r0 + j * zrows, zrows)])

        def edge_pass(s_ref, d_ref, table_ref, ept, nfull):
            base = s * ept

            def body(k, _):
                off = base + k * _CHUNK
                pltpu.sync_copy(s_ref.at[pl.ds(off, _CHUNK)], sidx)
                pltpu.sync_copy(d_ref.at[pl.ds(off, _CHUNK)], didx)
                pltpu.async_copy(table_ref.at[sidx], gbuf, sem).wait()
                pltpu.sync_copy(gbuf, acc.at[didx], add=True)
                return 0
            lax.fori_loop(0, nfull, body, 0)
            if tail:
                off = base + nfull * _CHUNK
                pltpu.sync_copy(s_ref.at[pl.ds(off, tail)], tsidx)
                pltpu.sync_copy(d_ref.at[pl.ds(off, tail)], tdidx)
                pltpu.async_copy(table_ref.at[tsidx], tgbuf, sem).wait()
                pltpu.sync_copy(tgbuf, acc.at[tdidx], add=True)

        def dense_out(nrm_ref, dst_ref, with_bias, r0, nr):
            for j in range(nr // zrows):
                c0 = r0 + j * zrows
                pltpu.sync_copy(acc.at[pl.ds(c0, zrows)], stage)
                pltpu.sync_copy(nrm_ref.at[pl.ds(c0, zrows)], nbuf)

                def row(r, _):
                    scale = nbuf[r, :]

                    def colg(gg, _):
                        v = stage[r, pl.ds(gg * _LANES, _LANES)] * scale
                        if with_bias:
                            v = v + bbuf[pl.ds(gg * _LANES, _LANES)]
                        stage[r, pl.ds(gg * _LANES, _LANES)] = v
                        return 0
                    lax.fori_loop(0, ng, colg, 0)
                    return 0
                lax.fori_loop(0, zrows, row, 0)
                pltpu.sync_copy(stage, dst_ref.at[pl.ds(c0, zrows)])

        def half_flow(table_ref, b_ref, hv_ref, rst_ref):
            pltpu.sync_copy(b_ref, bbuf)
            two_path(zero_acc)
            plsc.subcore_barrier()
            edge_pass(s0_ref, d0_ref, table_ref, ept0, nfull0)
            plsc.subcore_barrier()
            two_path(functools.partial(dense_out, nr0_ref, hv_ref, False))
            plsc.subcore_barrier()
            two_path(zero_acc)
            plsc.subcore_barrier()
            edge_pass(s1_ref, d1_ref, hv_ref, ept1, nfull1)
            plsc.subcore_barrier()
            two_path(functools.partial(dense_out, nr1_ref, rst_ref, True))

        @pl.when(c == 0)
        def _():
            half_flow(y0_ref, b0_ref, hv0_ref, rst0_ref)

        @pl.when(c == 1)
        def _():
            half_flow(y1_ref, b1_ref, hv1_ref, rst1_ref)

    return conv_k(y0, y1, src0, dst0, src1, dst1, nr0, nr1, b0, b1)


def kernel(feat, edge_index0, edge_index1, num_recv_dst, num_send_dst,
           weight, bias):
    n_src = feat.shape[0] // 2
    nv = n_src
    half = weight.shape[1] // 2

    feat_src = lax.dynamic_slice_in_dim(feat, num_recv_dst, n_src, axis=0)
    src0 = edge_index0[0]
    dst0 = edge_index0[1]
    src1 = edge_index1[0]
    dst1 = edge_index1[1]
    b0 = bias[:half]
    b1 = bias[half:]

    norm_l, norm_r0, norm_r1 = _deg_norms(src0, dst0, dst1, nv)
    y0, y1 = _project(feat_src, weight, norm_l[:, :1])
    _, _, rst0, rst1 = _spconv(y0, y1, src0, dst0, src1, dst1,
                               norm_r0, norm_r1, b0, b1)
    return jnp.concatenate([rst0, rst1], axis=1)


# speedup vs baseline: 4.4052x; 1.3888x over previous
import functools

import jax
import jax.numpy as jnp
from jax import lax
from jax.experimental import pallas as pl
from jax.experimental.pallas import tpu as pltpu
from jax.experimental.pallas import tpu_sc as plsc

_NC = 2
_NS = 16
_LANES = 16
_CD = 128
_CC = 64

_F32 = jnp.float32
_I32 = jnp.int32


def _rsqrt16(v):
    x = jnp.maximum(v, 1.0)
    i = lax.bitcast_convert_type(x, _I32)
    i = jnp.int32(0x5F3759DF) - lax.shift_right_arithmetic(i, 1)
    y = lax.bitcast_convert_type(i, _F32)
    for _ in range(3):
        y = y * (1.5 - 0.5 * x * y * y)
    return y


def _tile_rows(m, s, fn):
    b, e = divmod(m, _NS)
    if e == 0:
        fn(s * b, b)
    else:
        @pl.when(s < e)
        def _():
            fn(s * (b + 1), b + 1)

        @pl.when(s >= e)
        def _():
            fn(e + s * b, b)


def _deg_norms(s0r, d0r, d1r, nv):
    cr = s0r.shape[0]
    maxn = cr // _NS + (1 if cr % _NS else 0)
    assert nv % _NS == 0
    ept = nv // _NS
    mesh = plsc.VectorSubcoreMesh(
        core_axis_name="c", subcore_axis_name="s",
        num_cores=_NC, num_subcores=_NS)

    @functools.partial(
        pl.kernel,
        out_type=[jax.ShapeDtypeStruct((nv, _LANES), _F32)] * 3,
        mesh=mesh,
        scratch_types=[
            pltpu.VMEM_SHARED((nv, _LANES), _F32),
            pltpu.VMEM_SHARED((nv, _LANES), _F32),
            pltpu.VMEM((ept, _LANES), _F32),
            pltpu.VMEM((maxn, _CD), _I32),
            pltpu.VMEM((_CD, _LANES), _F32),
            pltpu.SemaphoreType.DMA,
        ],
        compiler_params=pltpu.CompilerParams(use_tc_tiling_on_sc=False),
    )
    def deg_k(s0_ref, d0_ref, d1_ref, nl_ref, nr0_ref, nr1_ref,
              acc_a, acc_b, ext, idxb, ones, ssem):
        c = lax.axis_index("c")
        s = lax.axis_index("s")

        def fill_ones(r, _):
            ones[r, :] = jnp.ones((_LANES,), _F32)
            return 0
        lax.fori_loop(0, _CD, fill_ones, 0)

        def fill_zero(r, _):
            ext[r, :] = jnp.zeros((_LANES,), _F32)
            return 0
        lax.fori_loop(0, ept, fill_zero, 0)

        for acc in (acc_a, acc_b):
            pltpu.sync_copy(ext, acc.at[pl.ds(s * ept, ept)])
        plsc.subcore_barrier()

        def scatter_ones(idx_ref, acc):
            def go(r0, n):
                pltpu.sync_copy(idx_ref.at[pl.ds(r0, n)],
                                idxb.at[pl.ds(0, n)])

                def issue(j, _):
                    pltpu.async_copy(ones, acc.at[idxb.at[j]], ssem,
                                     add=True)
                    return 0
                lax.fori_loop(0, n, issue, 0)

                def drain(j, _):
                    pltpu.make_async_copy(ones, acc.at[idxb.at[j]],
                                          ssem).wait()
                    return 0
                lax.fori_loop(0, n, drain, 0)
            _tile_rows(cr, s, go)

        @pl.when(c == 0)
        def _():
            scatter_ones(s0_ref, acc_a)
            scatter_ones(d0_ref, acc_b)

        @pl.when(c == 1)
        def _():
            scatter_ones(d1_ref, acc_a)

        plsc.subcore_barrier()

        def extract(acc, out_ref):
            r0 = s * ept
            pltpu.sync_copy(acc.at[pl.ds(r0, ept)], ext)

            def row(r, _):
                ext[r, :] = _rsqrt16(ext[r, :])
                return 0
            lax.fori_loop(0, ept, row, 0)
            pltpu.sync_copy(ext, out_ref.at[pl.ds(r0, ept)])

        @pl.when(c == 0)
        def _():
            extract(acc_a, nl_ref)
            extract(acc_b, nr0_ref)

        @pl.when(c == 1)
        def _():
            extract(acc_a, nr1_ref)

    return deg_k(s0r, d0r, d1r)


def _project(x, w, nl):
    n, d_in = x.shape
    d_out = w.shape[1]
    half = d_out // 2
    bm = 400
    assert n % bm == 0

    def body(x_ref, w_ref, s_ref, y0_ref, y1_ref):
        y = jnp.dot(x_ref[...], w_ref[...],
                    preferred_element_type=jnp.float32)
        y = y * s_ref[...]
        y0_ref[...] = y[:, :half]
        y1_ref[...] = y[:, half:]

    return pl.pallas_call(
        body,
        grid=(n // bm,),
        in_specs=[
            pl.BlockSpec((bm, d_in), lambda i: (i, 0)),
            pl.BlockSpec((d_in, d_out), lambda i: (0, 0)),
            pl.BlockSpec((bm, 1), lambda i: (i, 0)),
        ],
        out_specs=[
            pl.BlockSpec((bm, half), lambda i: (i, 0)),
            pl.BlockSpec((bm, half), lambda i: (i, 0)),
        ],
        out_shape=[jax.ShapeDtypeStruct((n, half), _F32)] * 2,
    )(x, w, nl)


def _spconv(y0, y1, s0r, d0r, s1r, d1r, nr0, nr1, b0, b1):
    nv, half = y0.shape
    cr0 = s0r.shape[0]
    cr1 = s1r.shape[0]
    maxn = max(cr0, cr1) // _NS + 1
    zrows = 80
    rpt = 640
    rlast = nv - rpt * (_NS - 1)
    assert rlast > 0 and rpt % zrows == 0 and rlast % zrows == 0
    ng = half // _LANES
    mesh = plsc.VectorSubcoreMesh(
        core_axis_name="c", subcore_axis_name="s",
        num_cores=_NC, num_subcores=_NS)

    @functools.partial(
        pl.kernel,
        out_type=[jax.ShapeDtypeStruct((nv, half), _F32)] * 4,
        mesh=mesh,
        scratch_types=[
            pltpu.VMEM_SHARED((nv, half), _F32),
            pltpu.VMEM((zrows, half), _F32),
            pltpu.VMEM((_CC, half), _F32),
            pltpu.VMEM((_CC, half), _F32),
            pltpu.VMEM((maxn, _CC), _I32),
            pltpu.VMEM((maxn, _CC), _I32),
            pltpu.VMEM((zrows, _LANES), _F32),
            pltpu.VMEM((half,), _F32),
            pltpu.SemaphoreType.DMA,
            pltpu.SemaphoreType.DMA,
            pltpu.SemaphoreType.DMA,
        ],
        compiler_params=pltpu.CompilerParams(use_tc_tiling_on_sc=False),
    )
    def conv_k(y0_ref, y1_ref, s0_ref, d0_ref, s1_ref, d1_ref,
               nr0_ref, nr1_ref, b0_ref, b1_ref,
               hv0_ref, hv1_ref, rst0_ref, rst1_ref,
               acc, stage, g0, g1, sidx, didx, nbuf, bbuf,
               gsem0, gsem1, ssem):
        c = lax.axis_index("c")
        s = lax.axis_index("s")

        def fill_stage_zero():
            def zr(r, _):
                def zc(g, _):
                    stage[r, pl.ds(g * _LANES, _LANES)] = jnp.zeros(
                        (_LANES,), _F32)
                    return 0
                lax.fori_loop(0, ng, zc, 0)
                return 0
            lax.fori_loop(0, zrows, zr, 0)

        def dense_partition(fn):
            @pl.when(s < _NS - 1)
            def _():
                fn(s * rpt, rpt)

            @pl.when(s == _NS - 1)
            def _():
                fn((_NS - 1) * rpt, rlast)

        def zero_acc(r0, nr):
            for j in range(nr // zrows):
                pltpu.sync_copy(stage, acc.at[pl.ds(r0 + j * zrows, zrows)])

        def edge_pass(src_ref, dst_ref, table_ref, cr):
            def gather_start(j, buf, sem):
                pltpu.async_copy(table_ref.at[sidx.at[j]], buf, sem)

            def gather_wait(j, buf, sem):
                pltpu.make_async_copy(table_ref.at[sidx.at[j]], buf,
                                      sem).wait()

            def scat_start(j, buf):
                pltpu.async_copy(buf, acc.at[didx.at[j]], ssem, add=True)

            def scat_wait(j, buf):
                pltpu.make_async_copy(buf, acc.at[didx.at[j]], ssem).wait()

            def go(r0, n):
                pltpu.sync_copy(src_ref.at[pl.ds(r0, n)],
                                sidx.at[pl.ds(0, n)])
                pltpu.sync_copy(dst_ref.at[pl.ds(r0, n)],
                                didx.at[pl.ds(0, n)])
                gather_start(0, g0, gsem0)

                def body(k, _):
                    a = 2 * k
                    gather_wait(a, g0, gsem0)

                    @pl.when(k > 0)
                    def _():
                        scat_wait(a - 1, g1)
                    gather_start(a + 1, g1, gsem1)
                    scat_start(a, g0)
                    gather_wait(a + 1, g1, gsem1)
                    scat_wait(a, g0)

                    @pl.when(a + 2 < n)
                    def _():
                        gather_start(a + 2, g0, gsem0)
                    scat_start(a + 1, g1)
                    return 0
                lax.fori_loop(0, n // 2, body, 0)
                if n % 2:
                    last = n - 1
                    gather_wait(last, g0, gsem0)
                    scat_wait(last - 1, g1)
                    pltpu.sync_copy(g0, acc.at[didx.at[last]], add=True)
                else:
                    scat_wait(n - 1, g1)
            _tile_rows(cr, s, go)

        def dense_out(nrm_ref, dst_ref, with_bias, r0, nr):
            for j in range(nr // zrows):
                c0 = r0 + j * zrows
                pltpu.sync_copy(acc.at[pl.ds(c0, zrows)], stage)
                pltpu.sync_copy(nrm_ref.at[pl.ds(c0, zrows)], nbuf)

                def row(r, _):
                    scale = nbuf[r, :]

                    def colg(gg, _):
                        v = stage[r, pl.ds(gg * _LANES, _LANES)] * scale
                        if with_bias:
                            v = v + bbuf[pl.ds(gg * _LANES, _LANES)]
                        stage[r, pl.ds(gg * _LANES, _LANES)] = v
                        return 0
                    lax.fori_loop(0, ng, colg, 0)
                    return 0
                lax.fori_loop(0, zrows, row, 0)
                pltpu.sync_copy(stage, dst_ref.at[pl.ds(c0, zrows)])

        def half_flow(table_ref, b_ref, hv_ref, rst_ref):
            pltpu.sync_copy(b_ref, bbuf)
            fill_stage_zero()
            dense_partition(zero_acc)
            plsc.subcore_barrier()
            edge_pass(s0_ref, d0_ref, table_ref, cr0)
            plsc.subcore_barrier()
            dense_partition(functools.partial(dense_out, nr0_ref, hv_ref,
                                              False))
            plsc.subcore_barrier()
            fill_stage_zero()
            dense_partition(zero_acc)
            plsc.subcore_barrier()
            edge_pass(s1_ref, d1_ref, hv_ref, cr1)
            plsc.subcore_barrier()
            dense_partition(functools.partial(dense_out, nr1_ref, rst_ref,
                                              True))

        @pl.when(c == 0)
        def _():
            half_flow(y0_ref, b0_ref, hv0_ref, rst0_ref)

        @pl.when(c == 1)
        def _():
            half_flow(y1_ref, b1_ref, hv1_ref, rst1_ref)

    return conv_k(y0, y1, s0r, d0r, s1r, d1r, nr0, nr1, b0, b1)


def kernel(feat, edge_index0, edge_index1, num_recv_dst, num_send_dst,
           weight, bias):
    n_src = feat.shape[0] // 2
    nv = n_src
    half = weight.shape[1] // 2

    feat_src = lax.dynamic_slice_in_dim(feat, num_recv_dst, n_src, axis=0)
    src0 = edge_index0[0]
    dst0 = edge_index0[1]
    src1 = edge_index1[0]
    dst1 = edge_index1[1]
    b0 = bias[:half]
    b1 = bias[half:]

    norm_l, norm_r0, norm_r1 = _deg_norms(
        src0.reshape(-1, _CD), dst0.reshape(-1, _CD),
        dst1.reshape(-1, _CD), nv)
    y0, y1 = _project(feat_src, weight, norm_l[:, :1])
    _, _, rst0, rst1 = _spconv(
        y0, y1, src0.reshape(-1, _CC), dst0.reshape(-1, _CC),
        src1.reshape(-1, _CC), dst1.reshape(-1, _CC),
        norm_r0, norm_r1, b0, b1)
    return jnp.concatenate([rst0, rst1], axis=1)


# speedup vs baseline: 5.8605x; 1.3303x over previous
import functools

import jax
import jax.numpy as jnp
from jax import lax
from jax.experimental import pallas as pl
from jax.experimental.pallas import tpu as pltpu
from jax.experimental.pallas import tpu_sc as plsc

_NC = 2
_NS = 16
_LANES = 16
_CD = 128
_CC = 64

_F32 = jnp.float32
_I32 = jnp.int32


def _rsqrt16(v):
    x = jnp.maximum(v, 1.0)
    i = lax.bitcast_convert_type(x, _I32)
    i = jnp.int32(0x5F3759DF) - lax.shift_right_arithmetic(i, 1)
    y = lax.bitcast_convert_type(i, _F32)
    for _ in range(3):
        y = y * (1.5 - 0.5 * x * y * y)
    return y


def _tile_rows(m, s, fn):
    b, e = divmod(m, _NS)
    if e == 0:
        fn(s * b, b)
    else:
        @pl.when(s < e)
        def _():
            fn(s * (b + 1), b + 1)

        @pl.when(s >= e)
        def _():
            fn(e + s * b, b)


def _deg_norms(s0r, d0r, d1r, nv):
    cr = s0r.shape[0]
    maxn = cr // _NS + (1 if cr % _NS else 0)
    assert nv % _NS == 0
    ept = nv // _NS
    mesh = plsc.VectorSubcoreMesh(
        core_axis_name="c", subcore_axis_name="s",
        num_cores=_NC, num_subcores=_NS)

    @functools.partial(
        pl.kernel,
        out_type=[jax.ShapeDtypeStruct((nv, _LANES), _F32)] * 3,
        mesh=mesh,
        scratch_types=[
            pltpu.VMEM_SHARED((nv, _LANES), _F32),
            pltpu.VMEM_SHARED((nv, _LANES), _F32),
            pltpu.VMEM((ept, _LANES), _F32),
            pltpu.VMEM((maxn, _CD), _I32),
            pltpu.VMEM((_CD, _LANES), _F32),
            pltpu.SemaphoreType.DMA,
        ],
        compiler_params=pltpu.CompilerParams(use_tc_tiling_on_sc=False),
    )
    def deg_k(s0_ref, d0_ref, d1_ref, nl_ref, nr0_ref, nr1_ref,
              acc_a, acc_b, ext, idxb, ones, ssem):
        c = lax.axis_index("c")
        s = lax.axis_index("s")

        def fill_ones(r, _):
            ones[r, :] = jnp.ones((_LANES,), _F32)
            return 0
        lax.fori_loop(0, _CD, fill_ones, 0)

        def fill_zero(r, _):
            ext[r, :] = jnp.zeros((_LANES,), _F32)
            return 0
        lax.fori_loop(0, ept, fill_zero, 0)

        for acc in (acc_a, acc_b):
            pltpu.sync_copy(ext, acc.at[pl.ds(s * ept, ept)])
        plsc.subcore_barrier()

        def scatter_ones(idx_ref, acc):
            def go(r0, n):
                pltpu.sync_copy(idx_ref.at[pl.ds(r0, n)],
                                idxb.at[pl.ds(0, n)])

                def issue(j, _):
                    pltpu.async_copy(ones, acc.at[idxb.at[j]], ssem,
                                     add=True)
                    return 0
                lax.fori_loop(0, n, issue, 0)

                def drain(j, _):
                    pltpu.make_async_copy(ones, acc.at[idxb.at[j]],
                                          ssem).wait()
                    return 0
                lax.fori_loop(0, n, drain, 0)
            _tile_rows(cr, s, go)

        @pl.when(c == 0)
        def _():
            scatter_ones(s0_ref, acc_a)
            scatter_ones(d0_ref, acc_b)

        @pl.when(c == 1)
        def _():
            scatter_ones(d1_ref, acc_a)

        plsc.subcore_barrier()

        def extract(acc, out_ref):
            r0 = s * ept
            pltpu.sync_copy(acc.at[pl.ds(r0, ept)], ext)

            def row(r, _):
                ext[r, :] = _rsqrt16(ext[r, :])
                return 0
            lax.fori_loop(0, ept, row, 0)
            pltpu.sync_copy(ext, out_ref.at[pl.ds(r0, ept)])

        @pl.when(c == 0)
        def _():
            extract(acc_a, nl_ref)
            extract(acc_b, nr0_ref)

        @pl.when(c == 1)
        def _():
            extract(acc_a, nr1_ref)

    return deg_k(s0r, d0r, d1r)


def _project(x, w, nl):
    n, d_in = x.shape
    d_out = w.shape[1]
    half = d_out // 2
    bm = 400
    assert n % bm == 0

    def body(x_ref, w_ref, s_ref, y0_ref, y1_ref):
        y = jnp.dot(x_ref[...], w_ref[...],
                    preferred_element_type=jnp.float32)
        y = y * s_ref[...]
        y0_ref[...] = y[:, :half]
        y1_ref[...] = y[:, half:]

    return pl.pallas_call(
        body,
        grid=(n // bm,),
        in_specs=[
            pl.BlockSpec((bm, d_in), lambda i: (i, 0)),
            pl.BlockSpec((d_in, d_out), lambda i: (0, 0)),
            pl.BlockSpec((bm, 1), lambda i: (i, 0)),
        ],
        out_specs=[
            pl.BlockSpec((bm, half), lambda i: (i, 0)),
            pl.BlockSpec((bm, half), lambda i: (i, 0)),
        ],
        out_shape=[jax.ShapeDtypeStruct((n, half), _F32)] * 2,
    )(x, w, nl)


def _spconv(y0, y1, s0r, d0r, s1r, d1r, nr0, nr1, b0, b1):
    nv, half = y0.shape
    cr0 = s0r.shape[0]
    cr1 = s1r.shape[0]
    maxn = max(cr0, cr1) // _NS + 1
    zrows = 80
    rpt = 640
    rlast = nv - rpt * (_NS - 1)
    assert rlast > 0 and rpt % zrows == 0 and rlast % zrows == 0
    ng = half // _LANES
    mesh = plsc.VectorSubcoreMesh(
        core_axis_name="c", subcore_axis_name="s",
        num_cores=_NC, num_subcores=_NS)

    blk = 80

    @functools.partial(
        pl.kernel,
        out_type=[jax.ShapeDtypeStruct((nv, half), _F32)] * 2
        + [jax.ShapeDtypeStruct((nv, 2 * half), _F32)],
        mesh=mesh,
        scratch_types=[
            pltpu.VMEM_SHARED((nv, half), _F32),
            pltpu.VMEM((zrows, half), _F32),
            pltpu.VMEM((_CC, half), _F32),
            pltpu.VMEM((_CC, half), _F32),
            pltpu.VMEM((_CC, half), _F32),
            pltpu.VMEM((blk, _CC), _I32),
            pltpu.VMEM((blk, _CC), _I32),
            pltpu.VMEM((zrows, _LANES), _F32),
            pltpu.VMEM((half,), _F32),
            pltpu.SemaphoreType.DMA,
            pltpu.SemaphoreType.DMA,
            pltpu.SemaphoreType.DMA,
            pltpu.SemaphoreType.DMA,
        ],
        compiler_params=pltpu.CompilerParams(use_tc_tiling_on_sc=False),
    )
    def conv_k(y0_ref, y1_ref, s0_ref, d0_ref, s1_ref, d1_ref,
               nr0_ref, nr1_ref, b0_ref, b1_ref,
               hv0_ref, hv1_ref, rst_ref,
               acc, stage, g0, g1, g2, sidx, didx, nbuf, bbuf,
               gsem0, gsem1, gsem2, ssem):
        c = lax.axis_index("c")
        s = lax.axis_index("s")

        def fill_stage_zero():
            def zr(r, _):
                def zc(g, _):
                    stage[r, pl.ds(g * _LANES, _LANES)] = jnp.zeros(
                        (_LANES,), _F32)
                    return 0
                lax.fori_loop(0, ng, zc, 0)
                return 0
            lax.fori_loop(0, zrows, zr, 0)

        def dense_partition(fn):
            @pl.when(s < _NS - 1)
            def _():
                fn(s * rpt, rpt)

            @pl.when(s == _NS - 1)
            def _():
                fn((_NS - 1) * rpt, rlast)

        def zero_acc(r0, nr):
            for j in range(nr // zrows):
                pltpu.sync_copy(stage, acc.at[pl.ds(r0 + j * zrows, zrows)])

        def edge_pass(src_ref, dst_ref, table_ref, cr):
            bufs = ((g0, gsem0), (g1, gsem1), (g2, gsem2))

            def g_start(j, t):
                b, gs = bufs[t]
                pltpu.async_copy(table_ref.at[sidx.at[j]], b, gs)

            def g_wait(j, t):
                b, gs = bufs[t]
                pltpu.make_async_copy(table_ref.at[sidx.at[j]], b,
                                      gs).wait()

            def sc_start(j, t):
                pltpu.async_copy(bufs[t][0], acc.at[didx.at[j]], ssem,
                                 add=True)

            def sc_wait(j, t):
                pltpu.make_async_copy(bufs[t][0], acc.at[didx.at[j]],
                                      ssem).wait()

            def do_block(r0, n):
                pltpu.sync_copy(src_ref.at[pl.ds(r0, n)],
                                sidx.at[pl.ds(0, n)])
                pltpu.sync_copy(dst_ref.at[pl.ds(r0, n)],
                                didx.at[pl.ds(0, n)])
                g_start(0, 0)
                g_start(1, 1)

                def body(k, _):
                    for t in range(3):
                        j = 3 * k + t
                        g_wait(j, t)
                        if t == 0:
                            @pl.when(k > 0)
                            def _():
                                sc_wait(j - 1, 2)
                        else:
                            sc_wait(j - 1, t - 1)

                        @pl.when(j + 2 < n)
                        def _():
                            g_start(j + 2, (t + 2) % 3)
                        sc_start(j, t)
                    return 0
                lax.fori_loop(0, n // 3, body, 0)
                for j in range((n // 3) * 3, n):
                    g_wait(j, j % 3)
                    if j > 0:
                        sc_wait(j - 1, (j - 1) % 3)
                    sc_start(j, j % 3)
                sc_wait(n - 1, (n - 1) % 3)

            def go(r0, n):
                for b0_ in range(0, n, blk):
                    do_block(r0 + b0_, min(blk, n - b0_))
            _tile_rows(cr, s, go)

        def dense_out(nrm_ref, dst_slice, with_bias, r0, nr):
            for j in range(nr // zrows):
                c0 = r0 + j * zrows
                pltpu.sync_copy(acc.at[pl.ds(c0, zrows)], stage)
                pltpu.sync_copy(nrm_ref.at[pl.ds(c0, zrows)], nbuf)

                def row(r, _):
                    scale = nbuf[r, :]

                    def colg(gg, _):
                        v = stage[r, pl.ds(gg * _LANES, _LANES)] * scale
                        if with_bias:
                            v = v + bbuf[pl.ds(gg * _LANES, _LANES)]
                        stage[r, pl.ds(gg * _LANES, _LANES)] = v
                        return 0
                    lax.fori_loop(0, ng, colg, 0)
                    return 0
                lax.fori_loop(0, zrows, row, 0)
                pltpu.sync_copy(stage, dst_slice(c0))

        def half_flow(table_ref, b_ref, hv_ref, col0):
            pltpu.sync_copy(b_ref, bbuf)
            fill_stage_zero()
            dense_partition(zero_acc)
            plsc.subcore_barrier()
            edge_pass(s0_ref, d0_ref, table_ref, cr0)
            plsc.subcore_barrier()
            dense_partition(functools.partial(
                dense_out, nr0_ref,
                lambda c0_: hv_ref.at[pl.ds(c0_, zrows)], False))
            plsc.subcore_barrier()
            fill_stage_zero()
            dense_partition(zero_acc)
            plsc.subcore_barrier()
            edge_pass(s1_ref, d1_ref, hv_ref, cr1)
            plsc.subcore_barrier()
            dense_partition(functools.partial(
                dense_out, nr1_ref,
                lambda c0_: rst_ref.at[pl.ds(c0_, zrows),
                                       pl.ds(col0, half)], True))

        @pl.when(c == 0)
        def _():
            half_flow(y0_ref, b0_ref, hv0_ref, 0)

        @pl.when(c == 1)
        def _():
            half_flow(y1_ref, b1_ref, hv1_ref, half)

    return conv_k(y0, y1, s0r, d0r, s1r, d1r, nr0, nr1, b0, b1)


def kernel(feat, edge_index0, edge_index1, num_recv_dst, num_send_dst,
           weight, bias):
    n_src = feat.shape[0] // 2
    nv = n_src
    half = weight.shape[1] // 2

    feat_src = lax.dynamic_slice_in_dim(feat, num_recv_dst, n_src, axis=0)
    src0 = edge_index0[0]
    dst0 = edge_index0[1]
    src1 = edge_index1[0]
    dst1 = edge_index1[1]
    b0 = bias[:half]
    b1 = bias[half:]

    norm_l, norm_r0, norm_r1 = _deg_norms(
        src0.reshape(-1, _CD), dst0.reshape(-1, _CD),
        dst1.reshape(-1, _CD), nv)
    y0, y1 = _project(feat_src, weight, norm_l[:, :1])
    _, _, rst = _spconv(
        y0, y1, src0.reshape(-1, _CC), dst0.reshape(-1, _CC),
        src1.reshape(-1, _CC), dst1.reshape(-1, _CC),
        norm_r0, norm_r1, b0, b1)
    return rst
